# baseline (speedup 1.0000x reference)
import functools

import jax
import jax.numpy as jnp
from jax import lax
from jax.experimental import pallas as pl
from jax.experimental.pallas import tpu as pltpu
from jax.experimental.pallas import tpu_sc as plsc

N = 10000
E = 320000
D = 128
NC = 2
NS = 16
NW = NC * NS

CHUNK = 128
NCHUNKS = E // CHUNK
BASE_CH = NCHUNKS // NW
EXTRA = NCHUNKS - BASE_CH * NW
STRIPE = 624
REM_BASE = NS * STRIPE
REM = N - REM_BASE
USERS_PER_W = 2048 // NW

_mesh = plsc.VectorSubcoreMesh(core_axis_name="c", subcore_axis_name="s")


def _worker_id():
    return lax.axis_index("s") * NC + lax.axis_index("c")


def _striped(s, fn):
    fn(s * STRIPE, STRIPE)

    @pl.when(s == NS - 1)
    def _():
        fn(REM_BASE, REM)


HBUF = 6
HGROUPS = (2500 // NW + 1 + HBUF - 1) // HBUF


@functools.partial(
    pl.kernel,
    out_type=(
        jax.ShapeDtypeStruct((NC, N, 16), jnp.float32),
        jax.ShapeDtypeStruct((NC, N, 16), jnp.float32),
    ),
    mesh=_mesh,
    compiler_params=pltpu.CompilerParams(use_tc_tiling_on_sc=False),
    scratch_types=(
        [pltpu.VMEM_SHARED((N, 16), jnp.float32)] * 2
        + [pltpu.VMEM((CHUNK, 16), jnp.float32)]
        + [pltpu.VMEM((CHUNK,), jnp.int32)] * (2 * HBUF)
        + [pltpu.SemaphoreType.DMA] * (4 * HBUF)
    ),
)
def _hist_kernel(src_hbm, dst_hbm, zeros16_hbm, ones_hbm,
                 degs_out, degd_out, acc_s, acc_d, ones_v, *bufs):
    sidx = bufs[0:HBUF]
    didx = bufs[HBUF:2 * HBUF]
    sems = bufs[2 * HBUF:]
    sem_si = sems[0:HBUF]
    sem_di = sems[HBUF:2 * HBUF]
    sem_as = sems[2 * HBUF:3 * HBUF]
    sem_ad = sems[3 * HBUF:4 * HBUF]

    c = lax.axis_index("c")
    s = lax.axis_index("s")
    w = _worker_id()

    def _zero(r0, nr):
        pltpu.sync_copy(zeros16_hbm.at[pl.ds(r0, nr)], acc_s.at[pl.ds(r0, nr)])
        pltpu.sync_copy(zeros16_hbm.at[pl.ds(r0, nr)], acc_d.at[pl.ds(r0, nr)])

    _striped(s, _zero)
    pltpu.sync_copy(ones_hbm, ones_v)
    plsc.subcore_barrier()

    n_my = BASE_CH + (w < EXTRA).astype(jnp.int32)

    def _start_idx(b, q):
        e0 = (w + q * NW) * CHUNK
        pltpu.async_copy(src_hbm.at[pl.ds(e0, CHUNK)], sidx[b], sem_si[b])
        pltpu.async_copy(dst_hbm.at[pl.ds(e0, CHUNK)], didx[b], sem_di[b])

    for b in range(HBUF):
        _start_idx(b, b)

    @pl.loop(0, HGROUPS)
    def _(g):
        q0 = g * HBUF
        for b in range(HBUF):
            @pl.when(q0 + b < n_my)
            def _(b=b):
                pltpu.make_async_copy(src_hbm.at[pl.ds(0, CHUNK)], sidx[b],
                                      sem_si[b]).wait()
                pltpu.make_async_copy(dst_hbm.at[pl.ds(0, CHUNK)], didx[b],
                                      sem_di[b]).wait()
                pltpu.async_copy(ones_v, acc_s.at[sidx[b]], sem_as[b], add=True)
                pltpu.async_copy(ones_v, acc_d.at[didx[b]], sem_ad[b], add=True)
        for b in range(HBUF):
            @pl.when(q0 + b < n_my)
            def _(b=b):
                pltpu.make_async_copy(ones_v, acc_s.at[sidx[b]], sem_as[b]).wait()
                pltpu.make_async_copy(ones_v, acc_d.at[didx[b]], sem_ad[b]).wait()

            @pl.when(q0 + b + HBUF < n_my)
            def _(b=b):
                _start_idx(b, q0 + b + HBUF)

    plsc.subcore_barrier()

    def _wout(r0, nr):
        pltpu.sync_copy(acc_s.at[pl.ds(r0, nr)], degs_out.at[c, pl.ds(r0, nr)])
        pltpu.sync_copy(acc_d.at[pl.ds(r0, nr)], degd_out.at[c, pl.ds(r0, nr)])

    _striped(s, _wout)


NBUF = 3
NGROUPS = (BASE_CH + 1 + NBUF - 1) // NBUF + 1
NGROUPS += NGROUPS % 2


@functools.partial(
    pl.kernel,
    out_type=jax.ShapeDtypeStruct((NC, N, D), jnp.float32),
    mesh=_mesh,
    compiler_params=pltpu.CompilerParams(use_tc_tiling_on_sc=False),
    scratch_types=(
        [pltpu.VMEM_SHARED((N, D), jnp.float32)]
        + [pltpu.VMEM((2, CHUNK), jnp.int32)] * (2 * NBUF)
        + [pltpu.VMEM((CHUNK, D), jnp.float32)] * NBUF
        + [pltpu.SemaphoreType.DMA] * (4 * NBUF)
    ),
)
def _scatter_kernel(table_hbm, eidx_hbm, zerosd_hbm,
                    out_hbm, acc, *bufs):
    idx = (bufs[0:NBUF], bufs[NBUF:2 * NBUF])
    rows = bufs[2 * NBUF:3 * NBUF]
    sems = bufs[3 * NBUF:]
    sem_i = (sems[0:NBUF], sems[NBUF:2 * NBUF])
    sem_g = sems[2 * NBUF:3 * NBUF]
    sem_a = sems[3 * NBUF:4 * NBUF]

    c = lax.axis_index("c")
    s = lax.axis_index("s")
    w = _worker_id()

    def _zero(r0, nr):
        pltpu.sync_copy(zerosd_hbm.at[pl.ds(r0, nr)], acc.at[pl.ds(r0, nr)])

    _striped(s, _zero)
    plsc.subcore_barrier()

    n_my = BASE_CH + (w < EXTRA).astype(jnp.int32)

    def _start_idx(st, b, q):
        pltpu.async_copy(eidx_hbm.at[w + q * NW], idx[st][b], sem_i[st][b])

    def _wait_idx(st, b):
        pltpu.make_async_copy(eidx_hbm.at[0], idx[st][b], sem_i[st][b]).wait()

    for b in range(NBUF):
        _start_idx(0, b, b)

    def _group(g, st, nst):
        q0 = g * NBUF
        for b in range(NBUF):
            @pl.when(jnp.logical_and(q0 + b - NBUF >= 0, q0 + b - NBUF < n_my))
            def _(b=b, nst=nst):
                pltpu.make_async_copy(rows[b], acc.at[idx[nst][b].at[1]],
                                      sem_a[b]).wait()

            @pl.when(q0 + b < n_my)
            def _(b=b, st=st):
                _wait_idx(st, b)
                pltpu.async_copy(table_hbm.at[idx[st][b].at[0]], rows[b], sem_g[b])

            @pl.when(q0 + NBUF + b < n_my)
            def _(b=b, nst=nst):
                _start_idx(nst, b, q0 + NBUF + b)
        for b in range(NBUF):
            @pl.when(q0 + b < n_my)
            def _(b=b, st=st):
                pltpu.make_async_copy(table_hbm.at[idx[st][b].at[0]], rows[b],
                                      sem_g[b]).wait()
                pltpu.async_copy(rows[b], acc.at[idx[st][b].at[1]], sem_a[b],
                                 add=True)

    @pl.loop(0, NGROUPS // 2)
    def _(gg):
        _group(2 * gg, 0, 1)
        _group(2 * gg + 1, 1, 0)

    plsc.subcore_barrier()

    def _wout(r0, nr):
        pltpu.sync_copy(acc.at[pl.ds(r0, nr)], out_hbm.at[c, pl.ds(r0, nr)])

    _striped(s, _wout)


@functools.partial(
    pl.kernel,
    out_type=jax.ShapeDtypeStruct((2048, D), jnp.float32),
    mesh=_mesh,
    scratch_types=[
        pltpu.VMEM((USERS_PER_W,), jnp.int32),
        pltpu.VMEM((USERS_PER_W, D), jnp.float32),
        pltpu.SemaphoreType.DMA,
    ],
)
def _user_gather_kernel(h_hbm, users_hbm, out_hbm, uidx, rows, sem):
    w = _worker_id()
    base = w * USERS_PER_W
    pltpu.sync_copy(users_hbm.at[pl.ds(base, USERS_PER_W)], uidx)
    pltpu.async_copy(h_hbm.at[uidx], rows, sem).wait()
    pltpu.sync_copy(rows, out_hbm.at[pl.ds(base, USERS_PER_W)])


_BLK = 1000


def _mm_body(x_ref, w_ref, o_ref):
    o_ref[...] = jnp.dot(x_ref[...], w_ref[...],
                         preferred_element_type=jnp.float32)


def _mm(x, w):
    n = x.shape[0]
    return pl.pallas_call(
        _mm_body,
        grid=(n // _BLK,),
        in_specs=[
            pl.BlockSpec((_BLK, x.shape[1]), lambda i: (i, 0)),
            pl.BlockSpec(w.shape, lambda i: (0, 0)),
        ],
        out_specs=pl.BlockSpec((_BLK, w.shape[1]), lambda i: (i, 0)),
        out_shape=jax.ShapeDtypeStruct((n, w.shape[1]), jnp.float32),
    )(x, w)


def _norm_from(deg_ref):
    d = deg_ref[0, :, 0:1] + deg_ref[1, :, 0:1]
    return lax.rsqrt(jnp.maximum(d, 1.0))


_DEG_SPEC = pl.BlockSpec((NC, _BLK, 16), lambda i: (0, i, 0))


def _scale_body(hw_ref, degs_ref, o_ref):
    o_ref[...] = hw_ref[...] * _norm_from(degs_ref)


def _scale(hw, degs_p):
    return pl.pallas_call(
        _scale_body,
        grid=(N // _BLK,),
        in_specs=[
            pl.BlockSpec((_BLK, D), lambda i: (i, 0)),
            _DEG_SPEC,
        ],
        out_specs=pl.BlockSpec((_BLK, D), lambda i: (i, 0)),
        out_shape=jax.ShapeDtypeStruct((N, D), jnp.float32),
    )(hw, degs_p)


def _layer_mm_body(p_ref, degd_ref, degs_ref, b_ref, w_ref, o_ref):
    agg = p_ref[0] + p_ref[1]
    h = jax.nn.relu(agg * _norm_from(degd_ref) + b_ref[...])
    o_ref[...] = jnp.dot(h, w_ref[...],
                         preferred_element_type=jnp.float32) * _norm_from(degs_ref)


def _layer_mm(p, degd_p, degs_p, b, w):
    return pl.pallas_call(
        _layer_mm_body,
        grid=(N // _BLK,),
        in_specs=[
            pl.BlockSpec((NC, _BLK, D), lambda i: (0, i, 0)),
            _DEG_SPEC,
            _DEG_SPEC,
            pl.BlockSpec((1, D), lambda i: (0, 0)),
            pl.BlockSpec((D, D), lambda i: (0, 0)),
        ],
        out_specs=pl.BlockSpec((_BLK, D), lambda i: (i, 0)),
        out_shape=jax.ShapeDtypeStruct((N, D), jnp.float32),
    )(p, degd_p, degs_p, b, w)


def _layer_out_body(p_ref, degd_ref, b_ref, o_ref):
    agg = p_ref[0] + p_ref[1]
    o_ref[...] = jax.nn.relu(agg * _norm_from(degd_ref) + b_ref[...])


def _layer_out(p, degd_p, b):
    return pl.pallas_call(
        _layer_out_body,
        grid=(N // _BLK,),
        in_specs=[
            pl.BlockSpec((NC, _BLK, D), lambda i: (0, i, 0)),
            _DEG_SPEC,
            pl.BlockSpec((1, D), lambda i: (0, 0)),
        ],
        out_specs=pl.BlockSpec((_BLK, D), lambda i: (i, 0)),
        out_shape=jax.ShapeDtypeStruct((N, D), jnp.float32),
    )(p, degd_p, b)


def _mlp_body(uh_ref, w1_ref, b1_ref, w2_ref, b2_ref, o_ref):
    t = jnp.tanh(jnp.dot(uh_ref[...], w1_ref[...],
                         preferred_element_type=jnp.float32) + b1_ref[...])
    o_ref[...] = jnp.dot(t, w2_ref[...],
                         preferred_element_type=jnp.float32) + b2_ref[...]


def _mlp(uh, w1, b1, w2, b2):
    return pl.pallas_call(
        _mlp_body,
        out_shape=jax.ShapeDtypeStruct((uh.shape[0], w2.shape[1]), jnp.float32),
    )(uh, w1, b1, w2, b2)


def kernel(features, W1, b1, W2, b2, Ws1, bs1, Ws2, bs2, edge_index, users):
    src = edge_index[0].astype(jnp.int32)
    dst = edge_index[1].astype(jnp.int32)
    users = users.astype(jnp.int32)
    zerosd = jnp.zeros((N, D), jnp.float32)
    zeros16 = jnp.zeros((N, 16), jnp.float32)
    ones16 = jnp.ones((CHUNK, 16), jnp.float32)

    eidx = jnp.stack([src.reshape(NCHUNKS, CHUNK),
                      dst.reshape(NCHUNKS, CHUNK)], axis=1)

    degs_p, degd_p = _hist_kernel(src, dst, zeros16, ones16)
    hw1 = _mm(features, W1)
    scaled1 = _scale(hw1, degs_p)
    p1 = _scatter_kernel(scaled1, eidx, zerosd)
    scaled2 = _layer_mm(p1, degd_p, degs_p, b1.reshape(1, D), W2)
    p2 = _scatter_kernel(scaled2, eidx, zerosd)
    h = _layer_out(p2, degd_p, b2.reshape(1, D))
    uh = _user_gather_kernel(h, users)
    R = _mlp(uh, Ws1, bs1.reshape(1, -1), Ws2, bs2.reshape(1, -1))
    return (R, h)
